# baseline (speedup 1.0000x reference)
import functools

import jax
import jax.numpy as jnp
from jax import lax
from jax.experimental import pallas as pl
from jax.experimental.pallas import tpu as pltpu
from jax.experimental.pallas import tpu_sc as plsc

N = 10000
D = 128
R = 8
E = 320000

CHUNK = 64
NBUF = 4
CPW = 160
NW = 32
EPAD = NW * CPW * CHUNK
DBITS = 14
NPAD = 10240
RPT = NPAD // 16
DUMMY = N


def _transform_body(emb_ref, w_ref, b_ref, t_ref):
    x = emb_ref[...]
    w = w_ref[0]
    y = lax.dot_general(x, w, (((1,), (1,)), ((), ())),
                        preferred_element_type=jnp.float32)
    t_ref[0] = y + b_ref[0]


def _transform(emb, W, b):
    BN = 1000
    out = pl.pallas_call(
        _transform_body,
        grid=(R, N // BN),
        in_specs=[
            pl.BlockSpec((BN, D), lambda r, i: (i, 0)),
            pl.BlockSpec((1, D, D), lambda r, i: (r, 0, 0)),
            pl.BlockSpec((1, 1, D), lambda r, i: (r, 0, 0)),
        ],
        out_specs=pl.BlockSpec((1, BN, D), lambda r, i: (r, i, 0)),
        out_shape=jax.ShapeDtypeStruct((R, N, D), jnp.float32),
    )(emb, W, b.reshape(R, 1, D))
    return out.reshape(R * N, D)


def _pidx_body(et_ref, src_ref, dst_ref, p_ref):
    gidx = et_ref[...] * N + src_ref[...]
    p_ref[...] = gidx * (1 << DBITS) + dst_ref[...]


def _make_pidx(et_p, src_p, dst_p):
    rows = EPAD // 128
    BR = rows // 2
    out = pl.pallas_call(
        _pidx_body,
        grid=(2,),
        in_specs=[
            pl.BlockSpec((BR, 128), lambda i: (i, 0)),
            pl.BlockSpec((BR, 128), lambda i: (i, 0)),
            pl.BlockSpec((BR, 128), lambda i: (i, 0)),
        ],
        out_specs=pl.BlockSpec((BR, 128), lambda i: (i, 0)),
        out_shape=jax.ShapeDtypeStruct((rows, 128), jnp.int32),
    )(et_p.reshape(rows, 128), src_p.reshape(rows, 128), dst_p.reshape(rows, 128))
    return out.reshape(EPAD)


@functools.cache
def _sc_scatter_kernel():
    mesh = plsc.VectorSubcoreMesh(core_axis_name="c", subcore_axis_name="s")
    return functools.partial(
        pl.kernel,
        mesh=mesh,
        out_type=jax.ShapeDtypeStruct((2, NPAD, D), jnp.float32),
        scratch_types=[
            pltpu.VMEM((NBUF, CHUNK), jnp.int32),
            pltpu.VMEM((NBUF, CHUNK), jnp.int32),
            pltpu.VMEM((NBUF, CHUNK), jnp.int32),
            pltpu.VMEM((NBUF, CHUNK, D), jnp.float32),
            pltpu.VMEM_SHARED((NPAD, D), jnp.float32),
            pltpu.SemaphoreType.DMA,
            pltpu.SemaphoreType.DMA,
            pltpu.SemaphoreType.DMA,
            pltpu.SemaphoreType.DMA,
            pltpu.SemaphoreType.DMA,
            pltpu.SemaphoreType.DMA,
            pltpu.SemaphoreType.DMA,
            pltpu.SemaphoreType.DMA,
        ],
    )(_sc_scatter_body)


def _sc_scatter_body(pidx_hbm, t_hbm, out_hbm, pchunk, gch, dch, rows, acc,
                     isem0, isem1, isem2, isem3, sem0, sem1, sem2, sem3):
    c = lax.axis_index("c")
    s = lax.axis_index("s")
    wid = s * 2 + c
    row0 = s * RPT
    gsem = (sem0, sem1, sem2, sem3)
    isem = (isem0, isem1, isem2, isem3)

    def i_start(b, k):
        pltpu.async_copy(pidx_hbm.at[wid, k], pchunk.at[b], isem[b])

    def i_wait(b, k):
        pltpu.make_async_copy(pidx_hbm.at[wid, k], pchunk.at[b],
                              isem[b]).wait()

    for b in range(NBUF):
        i_start(b, b)

    zero16 = jnp.zeros((16,), jnp.float32)

    def zbody(i, carry):
        for j in range(8):
            rows[0, i, pl.ds(j * 16, 16)] = zero16
        return carry

    lax.fori_loop(0, CHUNK, zbody, 0)
    for t in range(RPT // CHUNK):
        pltpu.sync_copy(rows.at[0], acc.at[pl.ds(row0 + t * CHUNK, CHUNK)])
    plsc.subcore_barrier()

    dmask = jnp.full((16,), (1 << DBITS) - 1, jnp.int32)

    def unpack(b):
        for i in range(CHUNK // 16):
            sl = pl.ds(i * 16, 16)
            v = pchunk[b, sl]
            gch[b, sl] = lax.shift_right_logical(v, DBITS)
            dch[b, sl] = lax.bitwise_and(v, dmask)

    def g_start(b):
        pltpu.async_copy(t_hbm.at[gch.at[b]], rows.at[b], gsem[b])

    def g_wait(b):
        pltpu.make_async_copy(t_hbm.at[gch.at[b]], rows.at[b], gsem[b]).wait()

    def s_add(b):
        pltpu.sync_copy(rows.at[b], acc.at[dch.at[b]], add=True)

    for b in range(NBUF):
        i_wait(b, b)
        unpack(b)
        i_start(b, b + NBUF)
        g_start(b)

    def body(i, carry):
        k0 = i * NBUF
        for b in range(NBUF):
            g_wait(b)
            s_add(b)
            k = k0 + NBUF + b
            i_wait(b, k)
            unpack(b)
            i_start(b, jnp.minimum(k + NBUF, CPW - 1))
            g_start(b)
        return carry

    lax.fori_loop(0, CPW // NBUF - 1, body, 0)
    for b in range(NBUF):
        g_wait(b)
        s_add(b)
        i_wait(b, CPW - 1)
    plsc.subcore_barrier()

    pltpu.sync_copy(acc.at[pl.ds(row0, RPT)],
                    out_hbm.at[c, pl.ds(row0, RPT)])


def _combine_body(p0_ref, p1_ref, emb_ref, g_ref, bt_ref, out_ref):
    h = p0_ref[...] + p1_ref[...] + emb_ref[...]
    mu = jnp.mean(h, axis=1, keepdims=True)
    dlt = h - mu
    var = jnp.mean(dlt * dlt, axis=1, keepdims=True)
    out_ref[...] = dlt * lax.rsqrt(var + 1e-5) * g_ref[...] + bt_ref[...]


def _combine(p0, p1, emb, gamma, beta):
    BN = 1000
    return pl.pallas_call(
        _combine_body,
        grid=(N // BN,),
        in_specs=[
            pl.BlockSpec((BN, D), lambda i: (i, 0)),
            pl.BlockSpec((BN, D), lambda i: (i, 0)),
            pl.BlockSpec((BN, D), lambda i: (i, 0)),
            pl.BlockSpec((1, D), lambda i: (0, 0)),
            pl.BlockSpec((1, D), lambda i: (0, 0)),
        ],
        out_specs=pl.BlockSpec((BN, D), lambda i: (i, 0)),
        out_shape=jax.ShapeDtypeStruct((N, D), jnp.float32),
    )(p0, p1, emb, gamma.reshape(1, D), beta.reshape(1, D))


def kernel(entity_emb, edge_index, edge_type, W, b, gamma, beta):
    src = edge_index[0].astype(jnp.int32)
    dst = edge_index[1].astype(jnp.int32)
    et = edge_type.astype(jnp.int32)

    npad = EPAD - E
    src_p = jnp.concatenate([src, jnp.zeros((npad,), jnp.int32)])
    et_p = jnp.concatenate([et, jnp.zeros((npad,), jnp.int32)])
    dst_p = jnp.concatenate([dst, jnp.full((npad,), DUMMY, jnp.int32)])

    t_table = _transform(entity_emb, W, b)
    pidx = _make_pidx(et_p, src_p, dst_p)
    partials = _sc_scatter_kernel()(pidx.reshape(NW, CPW, CHUNK),
                                    t_table)
    return _combine(partials[0, :N], partials[1, :N], entity_emb, gamma, beta)


# speedup vs baseline: 1.4736x; 1.4736x over previous
import functools

import jax
import jax.numpy as jnp
from jax import lax
from jax.experimental import pallas as pl
from jax.experimental.pallas import tpu as pltpu
from jax.experimental.pallas import tpu_sc as plsc

N = 10000
D = 128
R = 8
E = 320000

CHUNK = 128
NBUF = 2
KS = 46
KF = 112
SLOW_C = 0
NW = 32
TCH = 16 * (KS + KF)
EPAD = TCH * CHUNK
DBITS = 14
NPAD = 10240
RPT = NPAD // 16
DUMMY = N


def _transform_body(emb_ref, w_ref, b_ref, t_ref):
    x = emb_ref[...]
    w = w_ref[0]
    y = lax.dot_general(x, w, (((1,), (1,)), ((), ())),
                        preferred_element_type=jnp.float32)
    t_ref[0] = y + b_ref[0]


def _transform(emb, W, b):
    BN = 1000
    out = pl.pallas_call(
        _transform_body,
        grid=(R, N // BN),
        in_specs=[
            pl.BlockSpec((BN, D), lambda r, i: (i, 0)),
            pl.BlockSpec((1, D, D), lambda r, i: (r, 0, 0)),
            pl.BlockSpec((1, 1, D), lambda r, i: (r, 0, 0)),
        ],
        out_specs=pl.BlockSpec((1, BN, D), lambda r, i: (r, i, 0)),
        out_shape=jax.ShapeDtypeStruct((R, N, D), jnp.float32),
    )(emb, W, b.reshape(R, 1, D))
    return out.reshape(R * N, D)


def _pidx_body(et_ref, src_ref, dst_ref, p_ref):
    gidx = et_ref[...] * N + src_ref[...]
    p_ref[...] = gidx * (1 << DBITS) + dst_ref[...]


def _make_pidx(et_p, src_p, dst_p):
    rows = EPAD // 128
    BR = rows // 2
    out = pl.pallas_call(
        _pidx_body,
        grid=(2,),
        in_specs=[
            pl.BlockSpec((BR, 128), lambda i: (i, 0)),
            pl.BlockSpec((BR, 128), lambda i: (i, 0)),
            pl.BlockSpec((BR, 128), lambda i: (i, 0)),
        ],
        out_specs=pl.BlockSpec((BR, 128), lambda i: (i, 0)),
        out_shape=jax.ShapeDtypeStruct((rows, 128), jnp.int32),
    )(et_p.reshape(rows, 128), src_p.reshape(rows, 128), dst_p.reshape(rows, 128))
    return out.reshape(EPAD)


@functools.cache
def _sc_scatter_kernel():
    mesh = plsc.VectorSubcoreMesh(core_axis_name="c", subcore_axis_name="s")
    return functools.partial(
        pl.kernel,
        mesh=mesh,
        out_type=jax.ShapeDtypeStruct((2, NPAD, D), jnp.float32),
        scratch_types=[
            pltpu.VMEM((NBUF, CHUNK), jnp.int32),
            pltpu.VMEM((NBUF, CHUNK), jnp.int32),
            pltpu.VMEM((NBUF, CHUNK), jnp.int32),
            pltpu.VMEM((NBUF, CHUNK, D), jnp.float32),
            pltpu.VMEM_SHARED((NPAD, D), jnp.float32),
            pltpu.SemaphoreType.DMA,
            pltpu.SemaphoreType.DMA,
            pltpu.SemaphoreType.DMA,
            pltpu.SemaphoreType.DMA,
        ],
    )(_sc_scatter_body)


def _sc_scatter_body(pidx_hbm, t_hbm, out_hbm, pchunk, gch, dch, rows, acc,
                     isem0, isem1, sem0, sem1):
    c = lax.axis_index("c")
    s = lax.axis_index("s")
    row0 = s * RPT
    gsem = (sem0, sem1)
    isem = (isem0, isem1)

    slow = c == SLOW_C
    cnt = jnp.where(slow, KS, KF)
    base = jnp.where(slow, s * KS, 16 * KS + s * KF)
    last = base + cnt - 1

    def i_start(b, k):
        pltpu.async_copy(pidx_hbm.at[k], pchunk.at[b], isem[b])

    def i_wait(b, k):
        pltpu.make_async_copy(pidx_hbm.at[k], pchunk.at[b], isem[b]).wait()

    for b in range(NBUF):
        i_start(b, base + b)

    zero16 = jnp.zeros((16,), jnp.float32)

    def zbody(i, carry):
        for j in range(8):
            rows[0, i, pl.ds(j * 16, 16)] = zero16
        return carry

    lax.fori_loop(0, CHUNK, zbody, 0)
    nfull, rem = RPT // CHUNK, RPT % CHUNK
    for t in range(nfull):
        pltpu.sync_copy(rows.at[0], acc.at[pl.ds(row0 + t * CHUNK, CHUNK)])
    if rem:
        pltpu.sync_copy(rows.at[0, pl.ds(0, rem)],
                        acc.at[pl.ds(row0 + nfull * CHUNK, rem)])
    plsc.subcore_barrier()

    dmask = jnp.full((16,), (1 << DBITS) - 1, jnp.int32)

    def unpack(b):
        for i in range(CHUNK // 16):
            sl = pl.ds(i * 16, 16)
            v = pchunk[b, sl]
            gch[b, sl] = lax.shift_right_logical(v, DBITS)
            dch[b, sl] = lax.bitwise_and(v, dmask)

    def g_start(b):
        pltpu.async_copy(t_hbm.at[gch.at[b]], rows.at[b], gsem[b])

    def g_wait(b):
        pltpu.make_async_copy(t_hbm.at[gch.at[b]], rows.at[b], gsem[b]).wait()

    def s_add(b):
        pltpu.sync_copy(rows.at[b], acc.at[dch.at[b]], add=True)

    for b in range(NBUF):
        i_wait(b, base + b)
        unpack(b)
        i_start(b, base + b + NBUF)
        g_start(b)

    def body(i, carry):
        k0 = base + i * NBUF
        for b in range(NBUF):
            g_wait(b)
            s_add(b)
            k = k0 + NBUF + b
            i_wait(b, k)
            unpack(b)
            i_start(b, jnp.minimum(k + NBUF, last))
            g_start(b)
        return carry

    lax.fori_loop(0, cnt // NBUF - 1, body, 0)
    for b in range(NBUF):
        g_wait(b)
        s_add(b)
        i_wait(b, last)
    plsc.subcore_barrier()

    pltpu.sync_copy(acc.at[pl.ds(row0, RPT)],
                    out_hbm.at[c, pl.ds(row0, RPT)])


def _combine_body(p0_ref, p1_ref, emb_ref, g_ref, bt_ref, out_ref):
    h = p0_ref[...] + p1_ref[...] + emb_ref[...]
    mu = jnp.mean(h, axis=1, keepdims=True)
    dlt = h - mu
    var = jnp.mean(dlt * dlt, axis=1, keepdims=True)
    out_ref[...] = dlt * lax.rsqrt(var + 1e-5) * g_ref[...] + bt_ref[...]


def _combine(p0, p1, emb, gamma, beta):
    BN = 1000
    return pl.pallas_call(
        _combine_body,
        grid=(N // BN,),
        in_specs=[
            pl.BlockSpec((BN, D), lambda i: (i, 0)),
            pl.BlockSpec((BN, D), lambda i: (i, 0)),
            pl.BlockSpec((BN, D), lambda i: (i, 0)),
            pl.BlockSpec((1, D), lambda i: (0, 0)),
            pl.BlockSpec((1, D), lambda i: (0, 0)),
        ],
        out_specs=pl.BlockSpec((BN, D), lambda i: (i, 0)),
        out_shape=jax.ShapeDtypeStruct((N, D), jnp.float32),
    )(p0, p1, emb, gamma.reshape(1, D), beta.reshape(1, D))


def kernel(entity_emb, edge_index, edge_type, W, b, gamma, beta):
    src = edge_index[0].astype(jnp.int32)
    dst = edge_index[1].astype(jnp.int32)
    et = edge_type.astype(jnp.int32)

    npad = EPAD - E
    src_p = jnp.concatenate([src, jnp.zeros((npad,), jnp.int32)])
    et_p = jnp.concatenate([et, jnp.zeros((npad,), jnp.int32)])
    dst_p = jnp.concatenate([dst, jnp.full((npad,), DUMMY, jnp.int32)])

    t_table = _transform(entity_emb, W, b)
    pidx = _make_pidx(et_p, src_p, dst_p)
    partials = _sc_scatter_kernel()(pidx.reshape(TCH, CHUNK),
                                    t_table)
    return _combine(partials[0, :N], partials[1, :N], entity_emb, gamma, beta)
